# baseline (speedup 1.0000x reference)
import functools
import math

import jax
import jax.numpy as jnp
from jax import lax
from jax.experimental import pallas as pl
from jax.experimental.pallas import tpu as pltpu

_NEG = -1000000.0
_SENT = -3.0e38


def _bdot(a, b, dims):
    return lax.dot_general(a.astype(jnp.bfloat16), b.astype(jnp.bfloat16),
                           dims, preferred_element_type=jnp.float32)


def _stat_select_body(vl_ref, q32_ref, skf_ref, wqs_ref, wks_ref, wqt_ref,
                      wv_ref, wo_ref, sel_ref, sw_ref, qt_ref, wvo_ref):
    q32 = q32_ref[...]
    qs = _bdot(q32, wqs_ref[...], (((1,), (0,)), ((), ())))
    ks = _bdot(skf_ref[...], wks_ref[...], (((1,), (0,)), ((), ())))
    rows = []
    for b in range(8):
        a = qs[b * 4:(b + 1) * 4]
        kb = ks[b * 64:(b + 1) * 64]
        rows.append(_bdot(a, kb, (((1,), (1,)), ((), ()))))
    scores = jnp.concatenate(rows, axis=0) / math.sqrt(128.0)

    col = lax.broadcasted_iota(jnp.int32, (32, 64), 1)
    vlvec = jnp.concatenate(
        [jnp.full((4, 1), vl_ref[b], jnp.int32) for b in range(8)], axis=0)
    scores = jnp.where(col < vlvec, scores, _NEG)

    arr = scores
    vals = []
    idxs = []
    for _ in range(8):
        m = jnp.max(arr, axis=1, keepdims=True)
        hit = arr >= m
        idx = jnp.min(jnp.where(hit, col, 64), axis=1, keepdims=True)
        vals.append(m)
        idxs.append(idx)
        arr = jnp.where(col == idx, _SENT, arr)
    valsc = jnp.concatenate(vals, axis=1)
    idxsc = jnp.concatenate(idxs, axis=1)

    rowb = lax.broadcasted_iota(jnp.int32, (32, 8), 0) // 4
    gsel = idxsc + rowb * 64

    e = jnp.exp(valsc - valsc[:, :1])
    sw = e / jnp.sum(e, axis=1, keepdims=True)

    sel_ref[...] = gsel
    rmod = lax.broadcasted_iota(jnp.int32, (32, 128), 0) % _PAIRS
    diag = jnp.zeros((32, 128), jnp.float32)
    for p in range(_PAIRS):
        parts = []
        if p:
            parts.append(jnp.zeros((32, 8 * p), jnp.float32))
        parts.append(sw)
        parts.append(jnp.zeros((32, 120 - 8 * p), jnp.float32))
        diag = jnp.where(rmod == p, jnp.concatenate(parts, axis=1), diag)
    sw_ref[...] = diag
    qt_ref[...] = _bdot(q32, wqt_ref[...], (((1,), (0,)), ((), ())))
    wvo_ref[...] = jnp.dot(wv_ref[...], wo_ref[...],
                           preferred_element_type=jnp.float32)


_PAIRS = 4
_NB = 8 * _PAIRS


def _topk_mask(s):
    ib = lax.bitcast_convert_type(s, jnp.int32)
    t = ib ^ ((ib >> 31) & jnp.int32(0x7FFFFFFF))
    lo = jnp.min(t, axis=1, keepdims=True)
    hi = jnp.max(t, axis=1, keepdims=True)

    def cnt_ge(mid):
        return jnp.sum(jnp.where(t >= mid, 1.0, 0.0), axis=1, keepdims=True)

    for _ in range(12):
        w = hi - lo
        q8 = lax.shift_right_logical(w, 3)
        mids = [lo + q8 * k for k in range(1, 8)]
        ges = [cnt_ge(m) >= 64.0 for m in mids]
        nlo = lo
        nhi = hi
        for m, ge in zip(mids, ges):
            nlo = jnp.where(ge, m, nlo)
        for m, ge in zip(reversed(mids), reversed(ges)):
            nhi = jnp.where(ge, nhi, m - 1)
        lo = nlo
        hi = nhi
    for _ in range(3):
        fl = (lo >> 1) + (hi >> 1) + (lo & hi & 1)
        mid = fl + ((lo ^ hi) & 1)
        ge = cnt_ge(mid) >= 64.0
        lo = jnp.where(ge, mid, lo)
        hi = jnp.where(ge, hi, mid - 1)
    return t >= lo


def _main_body(sel_ref, qt_ref, wkt_ref, swd_ref, *refs):
    krefs = refs[0:_NB]
    vrefs = refs[_NB:2 * _NB]
    wvo_ref = refs[2 * _NB]
    out_ref = refs[2 * _NB + 1]
    i = pl.program_id(0)

    qtb = qt_ref[pl.ds(i * _PAIRS, _PAIRS), :].astype(jnp.bfloat16)
    wktb = wkt_ref[...].astype(jnp.bfloat16)
    srows = []
    for j in range(_NB):
        ktb = lax.dot_general(krefs[j][...].astype(jnp.bfloat16), wktb,
                              (((1,), (0,)), ((), ())),
                              preferred_element_type=jnp.float32
                              ).astype(jnp.bfloat16)
        q = qtb[j // 8:j // 8 + 1]
        srows.append(lax.dot_general(q, ktb, (((1,), (1,)), ((), ())),
                                     preferred_element_type=jnp.float32))
    sraw = jnp.concatenate(srows, axis=0)

    mask = _topk_mask(sraw)
    s = sraw * jnp.float32(1.0 / math.sqrt(128.0))
    m = jnp.max(s, axis=1, keepdims=True)
    e = jnp.where(mask, jnp.exp(s - m), 0.0).astype(jnp.bfloat16)

    accs = [lax.dot_general(e[j:j + 1], vrefs[j][...].astype(jnp.bfloat16),
                            (((1,), (0,)), ((), ())),
                            preferred_element_type=jnp.float32)
            for j in range(_NB)]
    a = jnp.concatenate(accs, axis=0)

    ones = jnp.ones((1, 512), jnp.bfloat16)
    z_row = lax.dot_general(ones, e, (((1,), (1,)), ((), ())),
                            preferred_element_type=jnp.float32)
    swd = swd_ref[pl.ds(i * _PAIRS, _PAIRS), 0:_NB]
    msel = swd / jnp.broadcast_to(z_row, (_PAIRS, _NB))
    out = lax.dot_general(msel, a, (((1,), (0,)), ((), ())),
                          preferred_element_type=jnp.float32)
    out = jnp.dot(out, wvo_ref[...], preferred_element_type=jnp.float32)
    out_ref[...] = out[None]


def _stat_select(q32, skf, vl, wqs, wks, wqt, wv, wo, interpret=False):
    return pl.pallas_call(
        _stat_select_body,
        grid=(1,),
        in_specs=[
            pl.BlockSpec(memory_space=pltpu.SMEM),
            pl.BlockSpec((32, 128), lambda i: (0, 0)),
            pl.BlockSpec((512, 128), lambda i: (0, 0)),
            pl.BlockSpec((128, 128), lambda i: (0, 0)),
            pl.BlockSpec((128, 128), lambda i: (0, 0)),
            pl.BlockSpec((128, 128), lambda i: (0, 0)),
            pl.BlockSpec((128, 128), lambda i: (0, 0)),
            pl.BlockSpec((128, 128), lambda i: (0, 0)),
        ],
        out_specs=[
            pl.BlockSpec((32, 8), lambda i: (0, 0)),
            pl.BlockSpec((32, 128), lambda i: (0, 0)),
            pl.BlockSpec((32, 128), lambda i: (0, 0)),
            pl.BlockSpec((128, 128), lambda i: (0, 0)),
        ],
        out_shape=[
            jax.ShapeDtypeStruct((32, 8), jnp.int32),
            jax.ShapeDtypeStruct((32, 128), jnp.float32),
            jax.ShapeDtypeStruct((32, 128), jnp.float32),
            jax.ShapeDtypeStruct((128, 128), jnp.float32),
        ],
        interpret=interpret,
    )(vl, q32, skf, wqs, wks, wqt, wv, wo)


def _main_call(sel, swl, qt, wkt, tk2, v2, wvo, interpret=False):
    nsteps = 32 // _PAIRS

    def kspec(j):
        return pl.BlockSpec(
            (512, 128), lambda i, s, j=j: (s[i * _PAIRS + j // 8, j % 8], 0))

    grid_spec = pltpu.PrefetchScalarGridSpec(
        num_scalar_prefetch=1,
        grid=(nsteps,),
        in_specs=[pl.BlockSpec((32, 128), lambda i, s: (0, 0)),
                  pl.BlockSpec((128, 128), lambda i, s: (0, 0)),
                  pl.BlockSpec((32, 128), lambda i, s: (0, 0))]
        + [kspec(j) for j in range(_NB)]
        + [kspec(j) for j in range(_NB)]
        + [pl.BlockSpec((128, 128), lambda i, s: (0, 0))],
        out_specs=pl.BlockSpec((1, _PAIRS, 128), lambda i, s: (i, 0, 0)),
    )
    return pl.pallas_call(
        _main_body,
        grid_spec=grid_spec,
        out_shape=jax.ShapeDtypeStruct((nsteps, _PAIRS, 128), jnp.float32),
        interpret=interpret,
    )(sel, qt, wkt, swl, *([tk2] * _NB), *([v2] * _NB), wvo)


@functools.partial(jax.jit, static_argnames=("interpret",))
def kernel(queries, stat_keys, token_keys, values, stat_valid_lens,
           Wq_stat, Wq_token, Wk_stat, Wk_token, Wv, Wo, interpret=False):
    q32 = queries.reshape(32, 128)
    skf = stat_keys.reshape(512, 128)
    sel, swl, qt, wvo = _stat_select(q32, skf, stat_valid_lens, Wq_stat,
                                     Wk_stat, Wq_token, Wv, Wo,
                                     interpret=interpret)
    tk2 = token_keys.reshape(512 * 512, 128)
    v2 = values.reshape(512 * 512, 128)
    out = _main_call(sel, swl, qt, Wk_token, tk2, v2, wvo,
                     interpret=interpret)
    return out.reshape(8, 4, 128)


# speedup vs baseline: 1.0018x; 1.0018x over previous
import math

import jax
import jax.numpy as jnp
from jax import lax
from jax.experimental import pallas as pl
from jax.experimental.pallas import tpu as pltpu

_NEG = -1000000.0
_SENT = -3.0e38


def _bdot(a, b, dims):
    return lax.dot_general(a.astype(jnp.bfloat16), b.astype(jnp.bfloat16),
                           dims, preferred_element_type=jnp.float32)


def _stat_select_body(vl_ref, q32_ref, skf_ref, wqs_ref, wks_ref, wqt_ref,
                      wv_ref, wo_ref, sel_ref, sw_ref, qt_ref, wvo_ref):
    q32 = q32_ref[...]
    qs = _bdot(q32, wqs_ref[...], (((1,), (0,)), ((), ())))
    ks = _bdot(skf_ref[...], wks_ref[...], (((1,), (0,)), ((), ())))
    rows = []
    for b in range(8):
        a = qs[b * 4:(b + 1) * 4]
        kb = ks[b * 64:(b + 1) * 64]
        rows.append(_bdot(a, kb, (((1,), (1,)), ((), ()))))
    scores = jnp.concatenate(rows, axis=0) / math.sqrt(128.0)

    col = lax.broadcasted_iota(jnp.int32, (32, 64), 1)
    vlvec = jnp.concatenate(
        [jnp.full((4, 1), vl_ref[b], jnp.int32) for b in range(8)], axis=0)
    scores = jnp.where(col < vlvec, scores, _NEG)

    arr = scores
    vals = []
    idxs = []
    for _ in range(8):
        m = jnp.max(arr, axis=1, keepdims=True)
        hit = arr >= m
        idx = jnp.min(jnp.where(hit, col, 64), axis=1, keepdims=True)
        vals.append(m)
        idxs.append(idx)
        arr = jnp.where(col == idx, _SENT, arr)
    valsc = jnp.concatenate(vals, axis=1)
    idxsc = jnp.concatenate(idxs, axis=1)

    rowb = lax.broadcasted_iota(jnp.int32, (32, 8), 0) // 4
    gsel = idxsc + rowb * 64

    e = jnp.exp(valsc - valsc[:, :1])
    sw = e / jnp.sum(e, axis=1, keepdims=True)

    sel_ref[...] = gsel
    rmod = lax.broadcasted_iota(jnp.int32, (32, 128), 0) % _PAIRS
    diag = jnp.zeros((32, 128), jnp.float32)
    for p in range(_PAIRS):
        parts = []
        if p:
            parts.append(jnp.zeros((32, 8 * p), jnp.float32))
        parts.append(sw)
        parts.append(jnp.zeros((32, 120 - 8 * p), jnp.float32))
        diag = jnp.where(rmod == p, jnp.concatenate(parts, axis=1), diag)
    sw_ref[...] = diag
    qt_ref[...] = _bdot(q32, wqt_ref[...], (((1,), (0,)), ((), ())))
    wvo_ref[...] = jnp.dot(wv_ref[...], wo_ref[...],
                           preferred_element_type=jnp.float32)


_PAIRS = 4
_NB = 8 * _PAIRS


def _topk_mask(s):
    ib = lax.bitcast_convert_type(s, jnp.int32)
    t = ib ^ ((ib >> 31) & jnp.int32(0x7FFFFFFF))
    lo = jnp.min(t, axis=1, keepdims=True)
    hi = jnp.max(t, axis=1, keepdims=True)

    def cnt_ge(mid):
        return jnp.sum(jnp.where(t >= mid, 1.0, 0.0), axis=1, keepdims=True)

    for _ in range(12):
        w = hi - lo
        q8 = lax.shift_right_logical(w, 3)
        mids = [lo + q8 * k for k in range(1, 8)]
        ges = [cnt_ge(m) >= 64.0 for m in mids]
        nlo = lo
        nhi = hi
        for m, ge in zip(mids, ges):
            nlo = jnp.where(ge, m, nlo)
        for m, ge in zip(reversed(mids), reversed(ges)):
            nhi = jnp.where(ge, nhi, m - 1)
        lo = nlo
        hi = nhi
    for _ in range(3):
        fl = (lo >> 1) + (hi >> 1) + (lo & hi & 1)
        mid = fl + ((lo ^ hi) & 1)
        ge = cnt_ge(mid) >= 64.0
        lo = jnp.where(ge, mid, lo)
        hi = jnp.where(ge, hi, mid - 1)
    return t >= lo


def _main_body(sel_ref, qt_ref, wkt_ref, swd_ref, *refs):
    krefs = refs[0:_NB]
    vrefs = refs[_NB:2 * _NB]
    wvo_ref = refs[2 * _NB]
    out_ref = refs[2 * _NB + 1]
    i = pl.program_id(0)

    qtb = qt_ref[pl.ds(i * _PAIRS, _PAIRS), :].astype(jnp.bfloat16)
    wktb = wkt_ref[...].astype(jnp.bfloat16)
    srows = []
    for j in range(_NB):
        ktb = lax.dot_general(krefs[j][...].astype(jnp.bfloat16), wktb,
                              (((1,), (0,)), ((), ())),
                              preferred_element_type=jnp.float32
                              ).astype(jnp.bfloat16)
        q = qtb[j // 8:j // 8 + 1]
        srows.append(lax.dot_general(q, ktb, (((1,), (1,)), ((), ())),
                                     preferred_element_type=jnp.float32))
    sraw = jnp.concatenate(srows, axis=0)

    mask = _topk_mask(sraw)
    s = sraw * jnp.float32(1.0 / math.sqrt(128.0))
    m = jnp.max(s, axis=1, keepdims=True)
    e = jnp.where(mask, jnp.exp(s - m), 0.0).astype(jnp.bfloat16)

    accs = [lax.dot_general(e[j:j + 1], vrefs[j][...].astype(jnp.bfloat16),
                            (((1,), (0,)), ((), ())),
                            preferred_element_type=jnp.float32)
            for j in range(_NB)]
    a = jnp.concatenate(accs, axis=0)

    ones = jnp.ones((1, 512), jnp.bfloat16)
    z_row = lax.dot_general(ones, e, (((1,), (1,)), ((), ())),
                            preferred_element_type=jnp.float32)
    swd = swd_ref[pl.ds(i * _PAIRS, _PAIRS), 0:_NB]
    msel = swd / jnp.broadcast_to(z_row, (_PAIRS, _NB))
    out = lax.dot_general(msel, a, (((1,), (0,)), ((), ())),
                          preferred_element_type=jnp.float32)
    out = jnp.dot(out, wvo_ref[...], preferred_element_type=jnp.float32)
    out_ref[...] = out[None]


def _stat_select(q32, skf, vl, wqs, wks, wqt, wv, wo):
    return pl.pallas_call(
        _stat_select_body,
        grid=(1,),
        in_specs=[
            pl.BlockSpec(memory_space=pltpu.SMEM),
            pl.BlockSpec((32, 128), lambda i: (0, 0)),
            pl.BlockSpec((512, 128), lambda i: (0, 0)),
            pl.BlockSpec((128, 128), lambda i: (0, 0)),
            pl.BlockSpec((128, 128), lambda i: (0, 0)),
            pl.BlockSpec((128, 128), lambda i: (0, 0)),
            pl.BlockSpec((128, 128), lambda i: (0, 0)),
            pl.BlockSpec((128, 128), lambda i: (0, 0)),
        ],
        out_specs=[
            pl.BlockSpec((32, 8), lambda i: (0, 0)),
            pl.BlockSpec((32, 128), lambda i: (0, 0)),
            pl.BlockSpec((32, 128), lambda i: (0, 0)),
            pl.BlockSpec((128, 128), lambda i: (0, 0)),
        ],
        out_shape=[
            jax.ShapeDtypeStruct((32, 8), jnp.int32),
            jax.ShapeDtypeStruct((32, 128), jnp.float32),
            jax.ShapeDtypeStruct((32, 128), jnp.float32),
            jax.ShapeDtypeStruct((128, 128), jnp.float32),
        ],
    )(vl, q32, skf, wqs, wks, wqt, wv, wo)


def _main_call(sel, swl, qt, wkt, tk2, v2, wvo):
    nsteps = 32 // _PAIRS

    def kspec(j):
        return pl.BlockSpec(
            (512, 128), lambda i, s, j=j: (s[i * _PAIRS + j // 8, j % 8], 0))

    grid_spec = pltpu.PrefetchScalarGridSpec(
        num_scalar_prefetch=1,
        grid=(nsteps,),
        in_specs=[pl.BlockSpec((32, 128), lambda i, s: (0, 0)),
                  pl.BlockSpec((128, 128), lambda i, s: (0, 0)),
                  pl.BlockSpec((32, 128), lambda i, s: (0, 0))]
        + [kspec(j) for j in range(_NB)]
        + [kspec(j) for j in range(_NB)]
        + [pl.BlockSpec((128, 128), lambda i, s: (0, 0))],
        out_specs=pl.BlockSpec((1, _PAIRS, 128), lambda i, s: (i, 0, 0)),
    )
    return pl.pallas_call(
        _main_body,
        grid_spec=grid_spec,
        out_shape=jax.ShapeDtypeStruct((nsteps, _PAIRS, 128), jnp.float32),
    )(sel, qt, wkt, swl, *([tk2] * _NB), *([v2] * _NB), wvo)


@jax.jit
def kernel(queries, stat_keys, token_keys, values, stat_valid_lens,
           Wq_stat, Wq_token, Wk_stat, Wk_token, Wv, Wo):
    q32 = queries.reshape(32, 128)
    skf = stat_keys.reshape(512, 128)
    sel, swl, qt, wvo = _stat_select(q32, skf, stat_valid_lens, Wq_stat,
                                     Wk_stat, Wq_token, Wv, Wo)
    tk2 = token_keys.reshape(512 * 512, 128)
    v2 = values.reshape(512 * 512, 128)
    out = _main_call(sel, swl, qt, Wk_token, tk2, v2, wvo)
    return out.reshape(8, 4, 128)
